# baseline (speedup 1.0000x reference)
import functools

import jax
import jax.numpy as jnp
from jax import lax
from jax.experimental import pallas as pl
from jax.experimental.pallas import tpu as pltpu
from jax.experimental.pallas import tpu_sc as plsc

N_STARS = 1000000
BATCH = 16384
HIST = 200
EPS = 0.001

PAD_ROWS = 7816
PAD_N = PAD_ROWS * 128

N_TOTAL = BATCH * HIST
NW = 32
LANES = 16
HALVES = 2
HALF_BATCH = BATCH // HALVES
HALF_TOTAL = HALF_BATCH * HIST
VIEW_ROWS = HALF_TOTAL // 128
CROWS = 40
CHUNK = CROWS * 128
N_CHUNKS = HALF_TOTAL // CHUNK // NW


def _combine_body(h_ref, c_ref, o_ref):
    o_ref[...] = jnp.maximum(h_ref[...] + c_ref[...], EPS)


def _combine_tables(hard_p, corr_p):
    return pl.pallas_call(
        _combine_body,
        out_shape=jax.ShapeDtypeStruct((PAD_ROWS, 128), jnp.float32),
    )(hard_p, corr_p)


def _sc_body(freq_hbm, idx_hbm, delta_hbm, out_hbm,
             idx_v, f_v, d_v, o_v, spmem, stage_v,
             sem_i, sem_f, sem_g, sem_o):
    wid = lax.axis_index("s") * 2 + lax.axis_index("c")
    base = wid * N_CHUNKS

    sid = lax.axis_index("s")
    shard = PAD_N // 16
    piece = shard // 8
    for k in range(8):
        off = sid * shard + k * piece
        pltpu.sync_copy(delta_hbm.at[pl.ds(off, piece)], stage_v)
        pltpu.sync_copy(stage_v, spmem.at[pl.ds(off, piece)])
    plsc.subcore_barrier()

    def idx_copy(i, b):
        return pltpu.make_async_copy(
            idx_hbm.at[pl.ds((base + i) * CHUNK, CHUNK)], idx_v[b], sem_i[b])

    def f_copy(i, b):
        return pltpu.make_async_copy(
            freq_hbm.at[pl.ds((base + i) * CROWS, CROWS)], f_v[b], sem_f[b])

    def g_copy(b):
        return pltpu.make_async_copy(spmem.at[idx_v[b]], d_v[b], sem_g[b])

    def o_copy(i, b):
        return pltpu.make_async_copy(
            o_v[b], out_hbm.at[pl.ds((base + i) * CROWS, CROWS)], sem_o[b])

    def compute(b):
        @plsc.parallel_loop(0, CROWS, 1)
        def _(r):
            for j in range(8):
                c = j * LANES
                o_v[b][r, pl.ds(c, LANES)] = lax.rem(
                    f_v[b][r, pl.ds(c, LANES)],
                    d_v[b][pl.ds(r * 128 + c, LANES)])

    idx_copy(0, 0).start()
    f_copy(0, 0).start()
    idx_copy(1, 1).start()
    f_copy(1, 1).start()
    idx_copy(0, 0).wait()
    g_copy(0).start()

    def pair_step(p, carry):
        for b in (0, 1):
            i = 2 * p + b

            @pl.when(i + 1 < N_CHUNKS)
            def _():
                idx_copy(i + 1, 1 - b).wait()
                g_copy(1 - b).start()

            g_copy(b).wait()
            f_copy(i, b).wait()

            @pl.when(i >= 2)
            def _():
                o_copy(i - 2, b).wait()

            compute(b)
            o_copy(i, b).start()

            @pl.when(i + 2 < N_CHUNKS)
            def _():
                idx_copy(i + 2, b).start()
                f_copy(i + 2, b).start()
        return carry

    lax.fori_loop(0, N_CHUNKS // 2, pair_step, 0)
    o_copy(N_CHUNKS - 2, 0).wait()
    o_copy(N_CHUNKS - 1, 1).wait()


@functools.partial(
    pl.kernel,
    out_type=jax.ShapeDtypeStruct((VIEW_ROWS, 128), jnp.float32),
    mesh=plsc.VectorSubcoreMesh(core_axis_name="c", subcore_axis_name="s"),
    scratch_types=[
        [pltpu.VMEM((CHUNK,), jnp.int32)] * 2,
        [pltpu.VMEM((CROWS, 128), jnp.float32)] * 2,
        [pltpu.VMEM((CHUNK,), jnp.float32)] * 2,
        [pltpu.VMEM((CROWS, 128), jnp.float32)] * 2,
        pltpu.VMEM_SHARED((PAD_N,), jnp.float32),
        pltpu.VMEM((PAD_N // 128,), jnp.float32),
        [pltpu.SemaphoreType.DMA] * 2,
        [pltpu.SemaphoreType.DMA] * 2,
        [pltpu.SemaphoreType.DMA] * 2,
        [pltpu.SemaphoreType.DMA] * 2,
    ],
)
def _sc_lookup_rem(freq_hbm, idx_hbm, delta_hbm, out_hbm,
                   idx_v, f_v, d_v, o_v, spmem, stage_v,
                   sem_i, sem_f, sem_g, sem_o):
    _sc_body(freq_hbm, idx_hbm, delta_hbm, out_hbm,
             idx_v, f_v, d_v, o_v, spmem, stage_v,
             sem_i, sem_f, sem_g, sem_o)


def kernel(frequencies, star_indices, delta_nu_hard, delta_nu_corr):
    hard_p = jnp.pad(delta_nu_hard, (0, PAD_N - N_STARS)).reshape(PAD_ROWS, 128)
    corr_p = jnp.pad(delta_nu_corr, (0, PAD_N - N_STARS)).reshape(PAD_ROWS, 128)
    delta = _combine_tables(hard_p, corr_p).reshape(PAD_N)
    halves = []
    for h in range(HALVES):
        rows = slice(h * HALF_BATCH, (h + 1) * HALF_BATCH)
        freq_v = frequencies[rows].reshape(VIEW_ROWS, 128)
        idx_flat = star_indices[rows].reshape(HALF_TOTAL).astype(jnp.int32)
        out_v = _sc_lookup_rem(freq_v, idx_flat, delta)
        halves.append(out_v.reshape(HALF_BATCH, HIST))
    return jnp.concatenate(halves, axis=0)


# speedup vs baseline: 1.0161x; 1.0161x over previous
import functools

import jax
import jax.numpy as jnp
from jax import lax
from jax.experimental import pallas as pl
from jax.experimental.pallas import tpu as pltpu
from jax.experimental.pallas import tpu_sc as plsc

N_STARS = 1000000
BATCH = 16384
HIST = 200
EPS = 0.001

PAD_ROWS = 7816
PAD_N = PAD_ROWS * 128

N_TOTAL = BATCH * HIST
NW = 32
LANES = 16
VIEW_ROWS = N_TOTAL // 128
CROWS = 40
CHUNK = CROWS * 128
N_CHUNKS = N_TOTAL // CHUNK // NW


def _sc_body(freq_hbm, idx_hbm, hard_hbm, corr_hbm, out_hbm,
             idx_v, f_v, d_v, o_v, spmem,
             sem_i, sem_f, sem_g, sem_o):
    wid = lax.axis_index("s") * 2 + lax.axis_index("c")
    base = wid * N_CHUNKS

    sid = lax.axis_index("s")

    def stage_piece(off, size):
        pltpu.sync_copy(hard_hbm.at[pl.ds(off, size)],
                        d_v[0].at[pl.ds(0, size)])
        pltpu.sync_copy(corr_hbm.at[pl.ds(off, size)],
                        d_v[1].at[pl.ds(0, size)])

        @plsc.parallel_loop(0, size, LANES)
        def _(p):
            sl = pl.ds(p, LANES)
            d_v[0][sl] = jnp.maximum(d_v[0][sl] + d_v[1][sl], EPS)

        pltpu.sync_copy(d_v[0].at[pl.ds(0, size)], spmem.at[pl.ds(off, size)])

    def stage_shard(shard_base, sizes):
        off = shard_base
        for s in sizes:
            stage_piece(off, s)
            off = off + s

    @pl.when(sid < 15)
    def _():
        stage_shard(sid * 62528, [CHUNK] * 12 + [1088])

    @pl.when(sid == 15)
    def _():
        stage_shard(15 * 62528, [CHUNK] * 12 + [640])

    plsc.subcore_barrier()

    def idx_copy(i, b):
        return pltpu.make_async_copy(
            idx_hbm.at[pl.ds((base + i) * CHUNK, CHUNK)], idx_v[b], sem_i[b])

    def f_copy(i, b):
        return pltpu.make_async_copy(
            freq_hbm.at[pl.ds((base + i) * CROWS, CROWS)], f_v[b], sem_f[b])

    def g_copy(b):
        return pltpu.make_async_copy(spmem.at[idx_v[b]], d_v[b], sem_g[b])

    def o_copy(i, b):
        return pltpu.make_async_copy(
            o_v[b], out_hbm.at[pl.ds((base + i) * CROWS, CROWS)], sem_o[b])

    def compute(b):
        @plsc.parallel_loop(0, CROWS, 1, unroll=2)
        def _(r):
            for j in range(8):
                c = j * LANES
                o_v[b][r, pl.ds(c, LANES)] = lax.rem(
                    f_v[b][r, pl.ds(c, LANES)],
                    d_v[b][pl.ds(r * 128 + c, LANES)])

    idx_copy(0, 0).start()
    f_copy(0, 0).start()
    idx_copy(1, 1).start()
    f_copy(1, 1).start()
    idx_copy(0, 0).wait()
    g_copy(0).start()

    def pair_step(p, carry):
        for b in (0, 1):
            i = 2 * p + b

            @pl.when(i + 1 < N_CHUNKS)
            def _():
                idx_copy(i + 1, 1 - b).wait()
                g_copy(1 - b).start()

            g_copy(b).wait()
            f_copy(i, b).wait()

            @pl.when(i >= 2)
            def _():
                o_copy(i - 2, b).wait()

            compute(b)
            o_copy(i, b).start()

            @pl.when(i + 2 < N_CHUNKS)
            def _():
                idx_copy(i + 2, b).start()
                f_copy(i + 2, b).start()
        return carry

    lax.fori_loop(0, N_CHUNKS // 2, pair_step, 0)
    o_copy(N_CHUNKS - 2, 0).wait()
    o_copy(N_CHUNKS - 1, 1).wait()


@functools.partial(
    pl.kernel,
    out_type=jax.ShapeDtypeStruct((VIEW_ROWS, 128), jnp.float32),
    mesh=plsc.VectorSubcoreMesh(core_axis_name="c", subcore_axis_name="s"),
    scratch_types=[
        [pltpu.VMEM((CHUNK,), jnp.int32)] * 2,
        [pltpu.VMEM((CROWS, 128), jnp.float32)] * 2,
        [pltpu.VMEM((CHUNK,), jnp.float32)] * 2,
        [pltpu.VMEM((CROWS, 128), jnp.float32)] * 2,
        pltpu.VMEM_SHARED((N_STARS,), jnp.float32),
        [pltpu.SemaphoreType.DMA] * 2,
        [pltpu.SemaphoreType.DMA] * 2,
        [pltpu.SemaphoreType.DMA] * 2,
        [pltpu.SemaphoreType.DMA] * 2,
    ],
)
def _sc_lookup_rem(freq_hbm, idx_hbm, hard_hbm, corr_hbm, out_hbm,
                   idx_v, f_v, d_v, o_v, spmem,
                   sem_i, sem_f, sem_g, sem_o):
    _sc_body(freq_hbm, idx_hbm, hard_hbm, corr_hbm, out_hbm,
             idx_v, f_v, d_v, o_v, spmem,
             sem_i, sem_f, sem_g, sem_o)


def kernel(frequencies, star_indices, delta_nu_hard, delta_nu_corr):
    freq_v = frequencies.reshape(VIEW_ROWS, 128)
    idx_flat = star_indices.reshape(N_TOTAL).astype(jnp.int32)
    out_v = _sc_lookup_rem(freq_v, idx_flat, delta_nu_hard, delta_nu_corr)
    return out_v.reshape(BATCH, HIST)


# speedup vs baseline: 1.1374x; 1.1194x over previous
import functools

import jax
import jax.numpy as jnp
from jax import lax
from jax.experimental import pallas as pl
from jax.experimental.pallas import tpu as pltpu
from jax.experimental.pallas import tpu_sc as plsc

N_STARS = 1000000
BATCH = 16384
HIST = 200
EPS = 0.001

PAD_ROWS = 7816
PAD_N = PAD_ROWS * 128

N_TOTAL = BATCH * HIST
NW = 32
LANES = 16
VIEW_ROWS = N_TOTAL // 128
CROWS = 40
CHUNK = CROWS * 128
N_CHUNKS = N_TOTAL // CHUNK // NW


def _combine_body(h_ref, c_ref, o_ref):
    o_ref[...] = jnp.maximum(h_ref[...] + c_ref[...], EPS)


def _combine_tables(hard_p, corr_p):
    return pl.pallas_call(
        _combine_body,
        out_shape=jax.ShapeDtypeStruct((PAD_ROWS, 128), jnp.float32),
    )(hard_p, corr_p)


def _sc_body(freq_hbm, idx_hbm, delta_hbm, out_hbm,
             idx_v, f_v, d_v, o_v, spmem, stage_v,
             sem_i, sem_f, sem_g, sem_o):
    wid = lax.axis_index("s") * 2 + lax.axis_index("c")
    base = wid * N_CHUNKS

    sid = lax.axis_index("s")
    shard = PAD_N // 16
    piece = shard // 8
    for k in range(8):
        off = sid * shard + k * piece
        pltpu.sync_copy(delta_hbm.at[pl.ds(off, piece)], stage_v)
        pltpu.sync_copy(stage_v, spmem.at[pl.ds(off, piece)])
    plsc.subcore_barrier()

    def idx_copy(i, b):
        return pltpu.make_async_copy(
            idx_hbm.at[pl.ds((base + i) * CHUNK, CHUNK)], idx_v[b], sem_i[b])

    def f_copy(i, b):
        return pltpu.make_async_copy(
            freq_hbm.at[pl.ds((base + i) * CROWS, CROWS)], f_v[b], sem_f[b])

    def g_copy(b):
        return pltpu.make_async_copy(spmem.at[idx_v[b]], d_v[b], sem_g[b])

    def o_copy(i, b):
        return pltpu.make_async_copy(
            o_v[b], out_hbm.at[pl.ds((base + i) * CROWS, CROWS)], sem_o[b])

    def compute(b):
        @plsc.parallel_loop(0, CROWS, 1, unroll=2)
        def _(r):
            for j in range(8):
                c = j * LANES
                o_v[b][r, pl.ds(c, LANES)] = lax.rem(
                    f_v[b][r, pl.ds(c, LANES)],
                    d_v[b][pl.ds(r * 128 + c, LANES)])

    idx_copy(0, 0).start()
    f_copy(0, 0).start()
    idx_copy(1, 1).start()
    f_copy(1, 1).start()
    idx_copy(0, 0).wait()
    g_copy(0).start()

    def pair_step(p, carry):
        for b in (0, 1):
            i = 2 * p + b

            @pl.when(i + 1 < N_CHUNKS)
            def _():
                idx_copy(i + 1, 1 - b).wait()
                g_copy(1 - b).start()

            g_copy(b).wait()
            f_copy(i, b).wait()

            @pl.when(i >= 2)
            def _():
                o_copy(i - 2, b).wait()

            compute(b)
            o_copy(i, b).start()

            @pl.when(i + 2 < N_CHUNKS)
            def _():
                idx_copy(i + 2, b).start()
                f_copy(i + 2, b).start()
        return carry

    lax.fori_loop(0, N_CHUNKS // 2, pair_step, 0)
    o_copy(N_CHUNKS - 2, 0).wait()
    o_copy(N_CHUNKS - 1, 1).wait()


@functools.partial(
    pl.kernel,
    out_type=jax.ShapeDtypeStruct((VIEW_ROWS, 128), jnp.float32),
    mesh=plsc.VectorSubcoreMesh(core_axis_name="c", subcore_axis_name="s"),
    scratch_types=[
        [pltpu.VMEM((CHUNK,), jnp.int32)] * 2,
        [pltpu.VMEM((CROWS, 128), jnp.float32)] * 2,
        [pltpu.VMEM((CHUNK,), jnp.float32)] * 2,
        [pltpu.VMEM((CROWS, 128), jnp.float32)] * 2,
        pltpu.VMEM_SHARED((PAD_N,), jnp.float32),
        pltpu.VMEM((PAD_N // 128,), jnp.float32),
        [pltpu.SemaphoreType.DMA] * 2,
        [pltpu.SemaphoreType.DMA] * 2,
        [pltpu.SemaphoreType.DMA] * 2,
        [pltpu.SemaphoreType.DMA] * 2,
    ],
)
def _sc_lookup_rem(freq_hbm, idx_hbm, delta_hbm, out_hbm,
                   idx_v, f_v, d_v, o_v, spmem, stage_v,
                   sem_i, sem_f, sem_g, sem_o):
    _sc_body(freq_hbm, idx_hbm, delta_hbm, out_hbm,
             idx_v, f_v, d_v, o_v, spmem, stage_v,
             sem_i, sem_f, sem_g, sem_o)


def kernel(frequencies, star_indices, delta_nu_hard, delta_nu_corr):
    hard_p = jnp.pad(delta_nu_hard, (0, PAD_N - N_STARS)).reshape(PAD_ROWS, 128)
    corr_p = jnp.pad(delta_nu_corr, (0, PAD_N - N_STARS)).reshape(PAD_ROWS, 128)
    delta = _combine_tables(hard_p, corr_p).reshape(PAD_N)
    freq_v = frequencies.reshape(VIEW_ROWS, 128)
    idx_flat = star_indices.reshape(N_TOTAL).astype(jnp.int32)
    out_v = _sc_lookup_rem(freq_v, idx_flat, delta)
    return out_v.reshape(BATCH, HIST)


# speedup vs baseline: 1.1689x; 1.0276x over previous
import functools

import jax
import jax.numpy as jnp
from jax import lax
from jax.experimental import pallas as pl
from jax.experimental.pallas import tpu as pltpu
from jax.experimental.pallas import tpu_sc as plsc

N_STARS = 1000000
BATCH = 16384
HIST = 200
EPS = 0.001

PAD_ROWS = 7816
PAD_N = PAD_ROWS * 128

N_TOTAL = BATCH * HIST
NW = 32
LANES = 16
VIEW_ROWS = N_TOTAL // 128
CROWS = 40
CHUNK = CROWS * 128
N_CHUNKS = N_TOTAL // CHUNK // NW


def _combine_body(h_ref, c_ref, o_ref):
    o_ref[...] = jnp.maximum(h_ref[...] + c_ref[...], EPS)


def _combine_tables(hard_p, corr_p):
    return pl.pallas_call(
        _combine_body,
        out_shape=jax.ShapeDtypeStruct((PAD_ROWS, 128), jnp.float32),
    )(hard_p, corr_p)


def _sc_body(freq_hbm, idx_hbm, delta_hbm, out_hbm,
             idx_v, f_v, d_v, o_v, spmem, stage_v,
             sem_i, sem_f, sem_g, sem_o):
    wid = lax.axis_index("s") * 2 + lax.axis_index("c")
    base = wid * N_CHUNKS

    sid = lax.axis_index("s")
    shard = PAD_N // 16
    piece = shard // 8
    for k in range(8):
        off = sid * shard + k * piece
        pltpu.sync_copy(delta_hbm.at[pl.ds(off, piece)], stage_v)
        pltpu.sync_copy(stage_v, spmem.at[pl.ds(off, piece)])
    plsc.subcore_barrier()

    def idx_copy(i, b):
        return pltpu.make_async_copy(
            idx_hbm.at[pl.ds((base + i) * CHUNK, CHUNK)], idx_v[b], sem_i[b])

    def f_copy(i, b):
        return pltpu.make_async_copy(
            freq_hbm.at[pl.ds((base + i) * CROWS, CROWS)], f_v[b], sem_f[b])

    def g_copy(b):
        return pltpu.make_async_copy(spmem.at[idx_v[b]], d_v[b], sem_g[b])

    def o_copy(i, b):
        return pltpu.make_async_copy(
            o_v[b], out_hbm.at[pl.ds((base + i) * CROWS, CROWS)], sem_o[b])

    def compute(b):
        @plsc.parallel_loop(0, CROWS, 1)
        def _(r):
            for j in range(8):
                c = j * LANES
                o_v[b][r, pl.ds(c, LANES)] = lax.rem(
                    f_v[b][r, pl.ds(c, LANES)],
                    d_v[b][pl.ds(r * 128 + c, LANES)])

    idx_copy(0, 0).start()
    f_copy(0, 0).start()
    idx_copy(1, 1).start()
    f_copy(1, 1).start()
    idx_copy(0, 0).wait()
    g_copy(0).start()

    def pair_step(p, carry):
        for b in (0, 1):
            i = 2 * p + b

            @pl.when(i + 1 < N_CHUNKS)
            def _():
                idx_copy(i + 1, 1 - b).wait()
                g_copy(1 - b).start()

            g_copy(b).wait()
            f_copy(i, b).wait()

            @pl.when(i >= 2)
            def _():
                o_copy(i - 2, b).wait()

            compute(b)
            o_copy(i, b).start()

            @pl.when(i + 2 < N_CHUNKS)
            def _():
                idx_copy(i + 2, b).start()
                f_copy(i + 2, b).start()
        return carry

    lax.fori_loop(0, N_CHUNKS // 2, pair_step, 0)
    o_copy(N_CHUNKS - 2, 0).wait()
    o_copy(N_CHUNKS - 1, 1).wait()


@functools.partial(
    pl.kernel,
    out_type=jax.ShapeDtypeStruct((VIEW_ROWS, 128), jnp.float32),
    mesh=plsc.VectorSubcoreMesh(core_axis_name="c", subcore_axis_name="s"),
    scratch_types=[
        [pltpu.VMEM((CHUNK,), jnp.int32)] * 2,
        [pltpu.VMEM((CROWS, 128), jnp.float32)] * 2,
        [pltpu.VMEM((CHUNK,), jnp.float32)] * 2,
        [pltpu.VMEM((CROWS, 128), jnp.float32)] * 2,
        pltpu.VMEM_SHARED((PAD_N,), jnp.float32),
        pltpu.VMEM((PAD_N // 128,), jnp.float32),
        [pltpu.SemaphoreType.DMA] * 2,
        [pltpu.SemaphoreType.DMA] * 2,
        [pltpu.SemaphoreType.DMA] * 2,
        [pltpu.SemaphoreType.DMA] * 2,
    ],
)
def _sc_lookup_rem(freq_hbm, idx_hbm, delta_hbm, out_hbm,
                   idx_v, f_v, d_v, o_v, spmem, stage_v,
                   sem_i, sem_f, sem_g, sem_o):
    _sc_body(freq_hbm, idx_hbm, delta_hbm, out_hbm,
             idx_v, f_v, d_v, o_v, spmem, stage_v,
             sem_i, sem_f, sem_g, sem_o)


def kernel(frequencies, star_indices, delta_nu_hard, delta_nu_corr):
    hard_p = jnp.pad(delta_nu_hard, (0, PAD_N - N_STARS)).reshape(PAD_ROWS, 128)
    corr_p = jnp.pad(delta_nu_corr, (0, PAD_N - N_STARS)).reshape(PAD_ROWS, 128)
    delta = _combine_tables(hard_p, corr_p).reshape(PAD_N)
    freq_v = frequencies.reshape(VIEW_ROWS, 128)
    idx_flat = star_indices.reshape(N_TOTAL).astype(jnp.int32)
    out_v = _sc_lookup_rem(freq_v, idx_flat, delta)
    return out_v.reshape(BATCH, HIST)
